# Initial kernel scaffold; baseline (speedup 1.0000x reference)
import jax
import jax.numpy as jnp
from jax.experimental import pallas as pl


def kernel(x_scalar, x_spherical, rbf, fcut, rsh, edge_index, W1, b1, W2, b2, Wr, br, ln_g, ln_b):
    raise NotImplementedError("write your pallas kernel here")


# speedup vs baseline: 2.8718x; 2.8718x over previous
import functools

import jax
import jax.numpy as jnp
from jax import lax
from jax.experimental import pallas as pl
from jax.experimental.pallas import tpu as pltpu
from jax.experimental.pallas import tpu_sc as plsc

N = 10000
E = 160000
D = 128
NB = 20
SDIM = 480
NIR = 224
HID = 576
EPS = 1e-5
GW = 896
CG = 128
NG = 5
CH = 128
NCH = E // CH
NW = 32
BN = 1000
BE = 1000


def _m3():
    c = lax.broadcasted_iota(jnp.int32, (64, 192), 0)
    r = lax.broadcasted_iota(jnp.int32, (64, 192), 1)
    return (r // 3 == c).astype(jnp.float32)


def _m5():
    c = lax.broadcasted_iota(jnp.int32, (32, 160), 0)
    r = lax.broadcasted_iota(jnp.int32, (32, 160), 1)
    return (r // 5 == c).astype(jnp.float32)


def _k1_body(xs_ref, xp_ref, w1_ref, b1_ref, w2_ref, b2_ref, g_ref, b_ref,
             gt_ref, i0_ref, i1_ref, i2_ref, i3_ref, i4_ref):
    xs = xs_ref[...]
    xp = xp_ref[...]
    mu = jnp.mean(xs, axis=-1, keepdims=True)
    xc = xs - mu
    var = jnp.mean(xc * xc, axis=-1, keepdims=True)
    s_in = xc / jnp.sqrt(var + EPS) * g_ref[...] + b_ref[...]
    s = xp[:, :128]
    v = xp[:, 128:320]
    t = xp[:, 320:480]
    s_mu = jnp.mean(s, axis=-1, keepdims=True)
    s_c = s - s_mu
    s_o = s_c / jnp.sqrt(jnp.mean(s_c * s_c, axis=-1, keepdims=True) + EPS)
    v_o = v / jnp.sqrt(3.0 * jnp.mean(v * v, axis=-1, keepdims=True) + EPS)
    t_o = t / jnp.sqrt(5.0 * jnp.mean(t * t, axis=-1, keepdims=True) + EPS)
    h = s_in @ w1_ref[...] + b1_ref[...]
    h = h * jax.nn.sigmoid(h)
    so = h @ w2_ref[...] + b2_ref[...]
    a_s = s_o * so[:, 0:128]
    a_v = v_o * (so[:, 128:192] @ _m3())
    a_t = t_o * (so[:, 192:224] @ _m5())
    zpad = jnp.zeros((xs.shape[0], 64), jnp.float32)
    gt_ref[...] = jnp.concatenate(
        [a_s, a_v, a_t, so[:, 224:448], so[:, 448:576], zpad], axis=-1)
    xcat = jnp.concatenate([xp, xs, zpad[:, :32]], axis=-1)
    i0_ref[...] = xcat[:, 0:128]
    i1_ref[...] = xcat[:, 128:256]
    i2_ref[...] = xcat[:, 256:384]
    i3_ref[...] = xcat[:, 384:512]
    i4_ref[...] = xcat[:, 512:640]


def _k3_body(g_ref, rbf_ref, fcut_ref, rsh_ref, wr_ref, br_ref,
             o0_ref, o1_ref, o2_ref, o3_ref, o4_ref):
    g = g_ref[...]
    rsh = rsh_ref[...]
    fw = (rbf_ref[...] @ wr_ref[...] + br_ref[...]) * fcut_ref[...]
    m3 = _m3()
    m5 = _m5()
    msg_s = g[:, 0:128] * fw[:, 0:128] + rsh[:, 0:128] * g[:, 480:608] * fw[:, 224:352]
    msg_v = g[:, 128:320] * (fw[:, 128:192] @ m3) \
        + rsh[:, 128:320] * ((g[:, 608:672] * fw[:, 352:416]) @ m3)
    msg_t = g[:, 320:480] * (fw[:, 192:224] @ m5) \
        + rsh[:, 320:480] * ((g[:, 672:704] * fw[:, 416:448]) @ m5)
    msg_b = g[:, 704:832] * fw[:, 448:576]
    zpad = jnp.zeros((g.shape[0], 32), jnp.float32)
    msg = jnp.concatenate([msg_s, msg_v, msg_t, msg_b, zpad], axis=-1)
    o0_ref[...] = msg[:, 0:128]
    o1_ref[...] = msg[:, 128:256]
    o2_ref[...] = msg[:, 256:384]
    o3_ref[...] = msg[:, 384:512]
    o4_ref[...] = msg[:, 512:640]


def _gather_body(gt, src, out, idx_v, buf, sem):
    wid = lax.axis_index("s") * 2 + lax.axis_index("c")
    nch = (NCH - 1 - wid) // NW + 1

    def body(k, carry):
        e0 = (wid + k * NW) * CH
        pltpu.sync_copy(src.at[pl.ds(e0, CH)], idx_v)
        pltpu.async_copy(gt.at[idx_v], buf, sem).wait()
        pltpu.sync_copy(buf, out.at[pl.ds(e0, CH)])
        return carry

    lax.fori_loop(0, nch, body, 0)


def _scatter_body(m0, m1, m2, m3_, m4, i0, i1, i2, i3, i4, dst,
                  o0, o1, o2, o3, o4, idx_v, mbuf, acc):
    cid = lax.axis_index("c")
    sid = lax.axis_index("s")
    nch = (NCH - 1 - sid) // 16 + 1
    r0 = sid * 624
    nr = 640

    def one_pass(msg, ini, out):
        pltpu.sync_copy(ini.at[pl.ds(r0, nr)], acc.at[pl.ds(r0, nr)])
        plsc.subcore_barrier()

        def body(k, carry):
            e0 = (sid + k * 16) * CH
            pltpu.sync_copy(dst.at[pl.ds(e0, CH)], idx_v)
            pltpu.sync_copy(msg.at[pl.ds(e0, CH)], mbuf)
            pltpu.sync_copy(mbuf, acc.at[idx_v], add=True)
            return carry

        lax.fori_loop(0, nch, body, 0)
        plsc.subcore_barrier()
        pltpu.sync_copy(acc.at[pl.ds(r0, nr)], out.at[pl.ds(r0, nr)])
        plsc.subcore_barrier()

    @pl.when(cid == 0)
    def _():
        one_pass(m0, i0, o0)
        one_pass(m1, i1, o1)

    @pl.when(cid == 1)
    def _():
        one_pass(m2, i2, o2)
        one_pass(m3_, i3, o3)
        one_pass(m4, i4, o4)


def kernel(x_scalar, x_spherical, rbf, fcut, rsh, edge_index,
           W1, b1, W2, b2, Wr, br, ln_g, ln_b):
    f32 = jnp.float32
    src = edge_index[1].astype(jnp.int32)
    dst = edge_index[0].astype(jnp.int32)

    nblk = N // BN
    g_table, i0, i1, i2, i3, i4 = pl.pallas_call(
        _k1_body,
        grid=(nblk,),
        in_specs=[
            pl.BlockSpec((BN, D), lambda i: (i, 0)),
            pl.BlockSpec((BN, SDIM), lambda i: (i, 0)),
            pl.BlockSpec((D, D), lambda i: (0, 0)),
            pl.BlockSpec((1, D), lambda i: (0, 0)),
            pl.BlockSpec((D, HID), lambda i: (0, 0)),
            pl.BlockSpec((1, HID), lambda i: (0, 0)),
            pl.BlockSpec((1, D), lambda i: (0, 0)),
            pl.BlockSpec((1, D), lambda i: (0, 0)),
        ],
        out_specs=[pl.BlockSpec((BN, GW), lambda i: (i, 0))]
        + [pl.BlockSpec((BN, CG), lambda i: (i, 0))] * NG,
        out_shape=[jax.ShapeDtypeStruct((N, GW), f32)]
        + [jax.ShapeDtypeStruct((N, CG), f32)] * NG,
    )(x_scalar, x_spherical, W1, b1.reshape(1, D), W2, b2.reshape(1, HID),
      ln_g.reshape(1, D), ln_b.reshape(1, D))

    mesh = plsc.VectorSubcoreMesh(core_axis_name="c", subcore_axis_name="s")
    gathered = pl.kernel(
        _gather_body,
        mesh=mesh,
        out_type=jax.ShapeDtypeStruct((E, GW), f32),
        scratch_types=[
            pltpu.VMEM((CH,), jnp.int32),
            pltpu.VMEM((CH, GW), f32),
            pltpu.SemaphoreType.DMA,
        ],
    )(g_table, src)

    eblk = E // BE
    m0, m1, m2, m3_, m4 = pl.pallas_call(
        _k3_body,
        grid=(eblk,),
        in_specs=[
            pl.BlockSpec((BE, GW), lambda i: (i, 0)),
            pl.BlockSpec((BE, NB), lambda i: (i, 0)),
            pl.BlockSpec((BE, 1), lambda i: (i, 0)),
            pl.BlockSpec((BE, SDIM), lambda i: (i, 0)),
            pl.BlockSpec((NB, HID), lambda i: (0, 0)),
            pl.BlockSpec((1, HID), lambda i: (0, 0)),
        ],
        out_specs=[pl.BlockSpec((BE, CG), lambda i: (i, 0))] * NG,
        out_shape=[jax.ShapeDtypeStruct((E, CG), f32)] * NG,
    )(gathered, rbf, fcut, rsh, Wr, br.reshape(1, HID))

    o0, o1, o2, o3, o4 = pl.kernel(
        _scatter_body,
        mesh=mesh,
        out_type=[jax.ShapeDtypeStruct((N, CG), f32)] * NG,
        scratch_types=[
            pltpu.VMEM((CH,), jnp.int32),
            pltpu.VMEM((CH, CG), f32),
            pltpu.VMEM_SHARED((N, CG), f32),
        ],
    )(m0, m1, m2, m3_, m4, i0, i1, i2, i3, i4, dst)

    out = jnp.concatenate([o0, o1, o2, o3, o4], axis=-1)
    return (out[:, SDIM:608], out[:, :SDIM])


# speedup vs baseline: 3.1503x; 1.0970x over previous
import functools

import jax
import jax.numpy as jnp
from jax import lax
from jax.experimental import pallas as pl
from jax.experimental.pallas import tpu as pltpu
from jax.experimental.pallas import tpu_sc as plsc

N = 10000
E = 160000
D = 128
NB = 20
SDIM = 480
NIR = 224
HID = 576
EPS = 1e-5
GW = 896
CG = 128
NG = 5
CH = 128
NCH = E // CH
CH2 = 64
NCH2 = E // CH2
TW2 = 80
TW4 = 80
NTRASH = 8
NW = 32
BN = 1000
BE = 1000


def _m3():
    c = lax.broadcasted_iota(jnp.int32, (64, 192), 0)
    r = lax.broadcasted_iota(jnp.int32, (64, 192), 1)
    return (r // 3 == c).astype(jnp.float32)


def _m5():
    c = lax.broadcasted_iota(jnp.int32, (32, 160), 0)
    r = lax.broadcasted_iota(jnp.int32, (32, 160), 1)
    return (r // 5 == c).astype(jnp.float32)


def _k1_body(xs_ref, xp_ref, w1_ref, b1_ref, w2_ref, b2_ref, g_ref, b_ref,
             gt_ref, i0_ref, i1_ref, i2_ref, i3_ref, i4_ref):
    xs = xs_ref[...]
    xp = xp_ref[...]
    mu = jnp.mean(xs, axis=-1, keepdims=True)
    xc = xs - mu
    var = jnp.mean(xc * xc, axis=-1, keepdims=True)
    s_in = xc / jnp.sqrt(var + EPS) * g_ref[...] + b_ref[...]
    s = xp[:, :128]
    v = xp[:, 128:320]
    t = xp[:, 320:480]
    s_mu = jnp.mean(s, axis=-1, keepdims=True)
    s_c = s - s_mu
    s_o = s_c / jnp.sqrt(jnp.mean(s_c * s_c, axis=-1, keepdims=True) + EPS)
    v_o = v / jnp.sqrt(3.0 * jnp.mean(v * v, axis=-1, keepdims=True) + EPS)
    t_o = t / jnp.sqrt(5.0 * jnp.mean(t * t, axis=-1, keepdims=True) + EPS)
    h = s_in @ w1_ref[...] + b1_ref[...]
    h = h * jax.nn.sigmoid(h)
    so = h @ w2_ref[...] + b2_ref[...]
    a_s = s_o * so[:, 0:128]
    a_v = v_o * (so[:, 128:192] @ _m3())
    a_t = t_o * (so[:, 192:224] @ _m5())
    zpad = jnp.zeros((xs.shape[0], 64), jnp.float32)
    gt_ref[...] = jnp.concatenate(
        [a_s, a_v, a_t, so[:, 224:448], so[:, 448:576], zpad], axis=-1)
    xcat = jnp.concatenate([xp, xs, zpad[:, :32]], axis=-1)
    i0_ref[...] = xcat[:, 0:128]
    i1_ref[...] = xcat[:, 128:256]
    i2_ref[...] = xcat[:, 256:384]
    i3_ref[...] = xcat[:, 384:512]
    i4_ref[...] = xcat[:, 512:640]


def _k3_body(g_ref, rbf_ref, fcut_ref, rsh_ref, wr_ref, br_ref,
             o0_ref, o1_ref, o2_ref, o3_ref, o4_ref):
    g = g_ref[...]
    rsh = rsh_ref[...]
    fw = (rbf_ref[...] @ wr_ref[...] + br_ref[...]) * fcut_ref[...]
    m3 = _m3()
    m5 = _m5()
    msg_s = g[:, 0:128] * fw[:, 0:128] + rsh[:, 0:128] * g[:, 480:608] * fw[:, 224:352]
    msg_v = g[:, 128:320] * (fw[:, 128:192] @ m3) \
        + rsh[:, 128:320] * ((g[:, 608:672] * fw[:, 352:416]) @ m3)
    msg_t = g[:, 320:480] * (fw[:, 192:224] @ m5) \
        + rsh[:, 320:480] * ((g[:, 672:704] * fw[:, 416:448]) @ m5)
    msg_b = g[:, 704:832] * fw[:, 448:576]
    zpad = jnp.zeros((g.shape[0], 32), jnp.float32)
    msg = jnp.concatenate([msg_s, msg_v, msg_t, msg_b, zpad], axis=-1)
    o0_ref[...] = msg[:, 0:128]
    o1_ref[...] = msg[:, 128:256]
    o2_ref[...] = msg[:, 256:384]
    o3_ref[...] = msg[:, 384:512]
    o4_ref[...] = msg[:, 512:640]


def _gather_body(gt, src2d, out, idx_all, buf0, buf1, g0, g1, w0, w1):
    wid = lax.axis_index("s") * 2 + lax.axis_index("c")
    base = wid * TW2
    pltpu.sync_copy(src2d.at[pl.ds(base, TW2)], idx_all)
    lastk = NCH2 - 1 - base

    def pair(j, carry):
        k0 = j * 2
        k1 = k0 + 1
        ka = jnp.minimum(k0, lastk)
        kb = jnp.minimum(k1, lastk)
        ea = (base + ka) * CH2
        eb = (base + kb) * CH2
        ha = pltpu.async_copy(gt.at[idx_all.at[ka]], buf0, g0)
        hb = pltpu.async_copy(gt.at[idx_all.at[kb]], buf1, g1)
        ha.wait()
        wa = pltpu.async_copy(buf0, out.at[pl.ds(ea, CH2)], w0)
        hb.wait()
        wb = pltpu.async_copy(buf1, out.at[pl.ds(eb, CH2)], w1)
        wa.wait()
        wb.wait()
        return carry

    lax.fori_loop(0, TW2 // 2, pair, 0)


def _scatter_body(m0, m1, m2, m3_, m4, i0, i1, i2, i3, i4, dst2d,
                  o0, o1, o2, o3, o4, idx_all, mb0, mb1, acc,
                  ms0, ms1, ss0, ss1):
    cid = lax.axis_index("c")
    sid = lax.axis_index("s")
    base = sid * TW4
    lastk = NCH - 1 - base
    r0 = sid * 624
    nr = 640
    pltpu.sync_copy(dst2d.at[pl.ds(base, TW4)], idx_all)

    def one_pass(msg, ini, out):
        pltpu.sync_copy(ini.at[pl.ds(r0, nr)], acc.at[pl.ds(r0, nr)])
        plsc.subcore_barrier()

        def pair(j, carry):
            k0 = j * 2
            k1 = k0 + 1
            ea = (base + jnp.minimum(k0, lastk)) * CH
            eb = (base + jnp.minimum(k1, lastk)) * CH
            ha = pltpu.async_copy(msg.at[pl.ds(ea, CH)], mb0, ms0)
            hb = pltpu.async_copy(msg.at[pl.ds(eb, CH)], mb1, ms1)
            ha.wait()
            sa = pltpu.async_copy(mb0, acc.at[idx_all.at[k0]], ss0, add=True)
            hb.wait()
            sb = pltpu.async_copy(mb1, acc.at[idx_all.at[k1]], ss1, add=True)
            sa.wait()
            sb.wait()
            return carry

        lax.fori_loop(0, TW4 // 2, pair, 0)
        plsc.subcore_barrier()
        pltpu.sync_copy(acc.at[pl.ds(r0, nr)], out.at[pl.ds(r0, nr)])
        plsc.subcore_barrier()

    @pl.when(cid == 0)
    def _():
        one_pass(m0, i0, o0)
        one_pass(m1, i1, o1)

    @pl.when(cid == 1)
    def _():
        one_pass(m2, i2, o2)
        one_pass(m3_, i3, o3)
        one_pass(m4, i4, o4)


def kernel(x_scalar, x_spherical, rbf, fcut, rsh, edge_index,
           W1, b1, W2, b2, Wr, br, ln_g, ln_b):
    f32 = jnp.float32
    src_i = edge_index[1].astype(jnp.int32)
    dst_i = edge_index[0].astype(jnp.int32)
    src2d = jnp.pad(src_i, (0, NW * TW2 * CH2 - E)).reshape(NW * TW2, CH2)
    dst2d = jnp.pad(dst_i, (0, 16 * TW4 * CH - E),
                    constant_values=N).reshape(16 * TW4, CH)

    nblk = N // BN
    g_table, i0, i1, i2, i3, i4 = pl.pallas_call(
        _k1_body,
        grid=(nblk,),
        in_specs=[
            pl.BlockSpec((BN, D), lambda i: (i, 0)),
            pl.BlockSpec((BN, SDIM), lambda i: (i, 0)),
            pl.BlockSpec((D, D), lambda i: (0, 0)),
            pl.BlockSpec((1, D), lambda i: (0, 0)),
            pl.BlockSpec((D, HID), lambda i: (0, 0)),
            pl.BlockSpec((1, HID), lambda i: (0, 0)),
            pl.BlockSpec((1, D), lambda i: (0, 0)),
            pl.BlockSpec((1, D), lambda i: (0, 0)),
        ],
        out_specs=[pl.BlockSpec((BN, GW), lambda i: (i, 0))]
        + [pl.BlockSpec((BN, CG), lambda i: (i, 0))] * NG,
        out_shape=[jax.ShapeDtypeStruct((N, GW), f32)]
        + [jax.ShapeDtypeStruct((N, CG), f32)] * NG,
    )(x_scalar, x_spherical, W1, b1.reshape(1, D), W2, b2.reshape(1, HID),
      ln_g.reshape(1, D), ln_b.reshape(1, D))

    mesh = plsc.VectorSubcoreMesh(core_axis_name="c", subcore_axis_name="s")
    gathered = pl.kernel(
        _gather_body,
        mesh=mesh,
        out_type=jax.ShapeDtypeStruct((E, GW), f32),
        scratch_types=[
            pltpu.VMEM((TW2, CH2), jnp.int32),
            pltpu.VMEM((CH2, GW), f32),
            pltpu.VMEM((CH2, GW), f32),
            pltpu.SemaphoreType.DMA,
            pltpu.SemaphoreType.DMA,
            pltpu.SemaphoreType.DMA,
            pltpu.SemaphoreType.DMA,
        ],
    )(g_table, src2d)

    eblk = E // BE
    m0, m1, m2, m3_, m4 = pl.pallas_call(
        _k3_body,
        grid=(eblk,),
        in_specs=[
            pl.BlockSpec((BE, GW), lambda i: (i, 0)),
            pl.BlockSpec((BE, NB), lambda i: (i, 0)),
            pl.BlockSpec((BE, 1), lambda i: (i, 0)),
            pl.BlockSpec((BE, SDIM), lambda i: (i, 0)),
            pl.BlockSpec((NB, HID), lambda i: (0, 0)),
            pl.BlockSpec((1, HID), lambda i: (0, 0)),
        ],
        out_specs=[pl.BlockSpec((BE, CG), lambda i: (i, 0))] * NG,
        out_shape=[jax.ShapeDtypeStruct((E, CG), f32)] * NG,
    )(gathered, rbf, fcut, rsh, Wr, br.reshape(1, HID))

    o0, o1, o2, o3, o4 = pl.kernel(
        _scatter_body,
        mesh=mesh,
        out_type=[jax.ShapeDtypeStruct((N, CG), f32)] * NG,
        scratch_types=[
            pltpu.VMEM((TW4, CH), jnp.int32),
            pltpu.VMEM((CH, CG), f32),
            pltpu.VMEM((CH, CG), f32),
            pltpu.VMEM_SHARED((N + NTRASH, CG), f32),
            pltpu.SemaphoreType.DMA,
            pltpu.SemaphoreType.DMA,
            pltpu.SemaphoreType.DMA,
            pltpu.SemaphoreType.DMA,
        ],
    )(m0, m1, m2, m3_, m4, i0, i1, i2, i3, i4, dst2d)

    out = jnp.concatenate([o0, o1, o2, o3, o4], axis=-1)
    return (out[:, SDIM:608], out[:, :SDIM])
